# baseline (speedup 1.0000x reference)
import jax
import jax.numpy as jnp
import numpy as np
from jax import lax
from jax.experimental import pallas as pl
from jax.experimental.pallas import tpu as pltpu
from jax.experimental.pallas import tpu_sc as plsc

N_U = 25000
N_I = 25000
D = 128
Q = 5
E = 400000
L = 2
B = 256
EPS = 0.05
TEMP = 0.2
LAMBDA_1 = 0.2

NC = 2
NS = 16
LANES = 16
HALF = D // 2

CHUNK = 128
E_PAD = 409600
EDGES_PER_TILE = E_PAD // NS
CHUNKS_PER_TILE = EDGES_PER_TILE // CHUNK

ROWS_PAD = 25088
ROWS_PER_TILE = ROWS_PAD // NS
HALF_ROWS = ROWS_PER_TILE // 2

QP = 16

_PERM = np.zeros((HALF,), np.int32)
for _g in range(HALF // 32):
    for _i in range(16):
        _PERM[_g * 32 + 2 * _i] = _g * 32 + _i
        _PERM[_g * 32 + 2 * _i + 1] = _g * 32 + 16 + _i

_mesh = plsc.VectorSubcoreMesh(
    core_axis_name="c", subcore_axis_name="s", num_cores=NC, num_subcores=NS)


def _lrelu(x):
    return jnp.where(x >= 0, x, 0.5 * x)


SUP = 2560
NSUP = EDGES_PER_TILE // SUP
SCH = 80
S_CH = SUP // SCH


def _spmm_body(tabA, tabB, gA, sA, vA, outA, outB, gall, sall, vall, idxG0,
               idxG1, sbuf0, sbuf1, rG0, rG1, rS0, rS1, gsem0, gsem1, ssem0,
               ssem1, zsem, acc):
    _spmm_phase(tabA, gA, sA, vA, outA, gall, sall, vall, idxG0, idxG1,
                sbuf0, sbuf1, rG0, rG1, rS0, rS1, gsem0, gsem1, ssem0,
                ssem1, zsem, acc)
    _spmm_phase(tabB, sA, gA, vA, outB, gall, sall, vall, idxG0, idxG1,
                sbuf0, sbuf1, rG0, rG1, rS0, rS1, gsem0, gsem1, ssem0,
                ssem1, zsem, acc)


def _spmm_phase(tab, g, s, v, out, gall, sall, vall, idxG0, idxG1, sbuf0,
                sbuf1, rG0, rG1, rS0, rS1, gsem0, gsem1, ssem0, ssem1,
                zsem, acc):
    c = lax.axis_index("c")
    sid = lax.axis_index("s")
    rG = (rG0, rG1)
    rS = (rS0, rS1)
    idxG = (idxG0, idxG1)
    sbuf = (sbuf0, sbuf1)
    gsem = (gsem0, gsem1)
    ssem = (ssem0, ssem1)

    zf = jnp.zeros((LANES,), jnp.float32)

    def zrow(i, carry):
        for kk in range(HALF // LANES):
            rS0[i, pl.ds(kk * LANES, LANES)] = zf
        return carry

    lax.fori_loop(0, SCH, zrow, 0)
    r0 = sid * ROWS_PER_TILE
    NZ = ROWS_PER_TILE // SCH
    remz = ROWS_PER_TILE % SCH
    for i in range(NZ):
        pltpu.async_copy(rS0, acc.at[pl.ds(r0 + i * SCH, SCH)], zsem)
    if remz:
        pltpu.async_copy(rS0.at[pl.ds(0, remz)],
                         acc.at[pl.ds(r0 + NZ * SCH, remz)], zsem)
    for i in range(NZ):
        pltpu.make_async_copy(rS0, acc.at[pl.ds(r0 + i * SCH, SCH)],
                              zsem).wait()
    if remz:
        pltpu.make_async_copy(rS0.at[pl.ds(0, remz)],
                              acc.at[pl.ds(r0 + NZ * SCH, remz)], zsem).wait()
    plsc.subcore_barrier()

    ebase = sid * EDGES_PER_TILE

    def mkidx(bi, cj):
        for kk in range(SCH // LANES):
            gv = gall[pl.ds(cj * SCH + kk * LANES, LANES)]
            idxG[bi][pl.ds(kk * LANES, LANES)] = gv * 2 + c

    def mksidx(bi, cj):
        for kk in range(SCH // LANES):
            sbuf[bi][pl.ds(kk * LANES, LANES)] = (
                sall[pl.ds(cj * SCH + kk * LANES, LANES)])

    def scale(bi, cj):
        def scale16(gg, inner):
            vv = vall[pl.ds(cj * SCH + gg * LANES, LANES)]
            for j in range(LANES):
                val = jnp.broadcast_to(vv[j], (LANES,))
                r = gg * LANES + j
                for kk in range(HALF // 32):
                    grp = rG[bi][r, pl.ds(kk * 32, 32)]
                    lo, hi = plsc.unpack(
                        grp, format=plsc.PackFormat.INTERLEAVED)
                    rS[bi][r, pl.ds(kk * 32, LANES)] = lo * val
                    rS[bi][r, pl.ds(kk * 32 + LANES, LANES)] = hi * val
            return inner

        lax.fori_loop(0, SCH // LANES, scale16, 0)

    def gdrain(bi):
        pltpu.make_async_copy(tab.at[pl.ds(0, SCH)], rG[bi], gsem[bi]).wait()

    def sdrain(bi):
        pltpu.make_async_copy(out.at[0, pl.ds(0, SCH)], rS[bi],
                              ssem[bi]).wait()

    def step(bi, ch, si, lookahead):
        gdrain(bi)

        @pl.when(si * S_CH + ch >= 2)
        def _():
            sdrain(bi)

        scale(bi, ch)
        mksidx(bi, ch)
        pltpu.async_copy(rS[bi], acc.at[sbuf[bi]], ssem[bi], add=True)
        if lookahead:
            mkidx(bi, ch + 2)
            pltpu.async_copy(tab.at[idxG[bi]], rG[bi], gsem[bi])

    def super_body(si, carry):
        base = ebase + si * SUP
        pltpu.sync_copy(g.at[pl.ds(base, SUP)], gall)
        pltpu.sync_copy(s.at[pl.ds(base, SUP)], sall)
        pltpu.sync_copy(v.at[pl.ds(base, SUP)], vall)

        for bi in range(2):
            mkidx(bi, bi)
            pltpu.async_copy(tab.at[idxG[bi]], rG[bi], gsem[bi])

        def pair(j2, inner):
            a = 2 * j2
            step(0, a, si, True)
            step(1, a + 1, si, True)
            return inner

        lax.fori_loop(0, S_CH // 2 - 1, pair, 0)
        step(0, S_CH - 2, si, False)
        step(1, S_CH - 1, si, False)
        return carry

    lax.fori_loop(0, NSUP, super_body, 0)
    sdrain(0)
    sdrain(1)
    plsc.subcore_barrier()

    pltpu.sync_copy(acc.at[pl.ds(r0, ROWS_PER_TILE)],
                    out.at[c, pl.ds(r0, ROWS_PER_TILE)])


_sc_params = pltpu.CompilerParams(
    use_tc_tiling_on_sc=False, needs_layout_passes=False)

_spmm = pl.kernel(
    _spmm_body,
    out_type=[jax.ShapeDtypeStruct((NC, ROWS_PAD, HALF), jnp.float32),
              jax.ShapeDtypeStruct((NC, ROWS_PAD, HALF), jnp.float32)],
    mesh=_mesh,
    compiler_params=_sc_params,
    scratch_types=[
        pltpu.VMEM((SUP,), jnp.int32),
        pltpu.VMEM((SUP,), jnp.int32),
        pltpu.VMEM((SUP,), jnp.float32),
        pltpu.VMEM((SCH,), jnp.int32),
        pltpu.VMEM((SCH,), jnp.int32),
        pltpu.VMEM((SCH,), jnp.int32),
        pltpu.VMEM((SCH,), jnp.int32),
        pltpu.VMEM((SCH, HALF), jnp.bfloat16),
        pltpu.VMEM((SCH, HALF), jnp.bfloat16),
        pltpu.VMEM((SCH, HALF), jnp.float32),
        pltpu.VMEM((SCH, HALF), jnp.float32),
        pltpu.SemaphoreType.DMA,
        pltpu.SemaphoreType.DMA,
        pltpu.SemaphoreType.DMA,
        pltpu.SemaphoreType.DMA,
        pltpu.SemaphoreType.DMA,
        pltpu.VMEM_SHARED((ROWS_PAD, HALF), jnp.float32),
    ],
)


NW = NC * NS
EPT2 = E_PAD // NW
FCH = EPT2 // CHUNK


def _filter_body(fu, fi, srcr, dstr, valr,
                 ug, us, uv, ucnt, ig, isl, iv, icnt,
                 sall2, dall2, vall2, sbufc, dbufc, fbU, fbI,
                 cgU, csU, cvU, cgI, csI, cvI, kvec, fsem):
    c = lax.axis_index("c")
    sid = lax.axis_index("s")
    w = sid * NC + c
    base = w * EPT2
    pltpu.sync_copy(srcr.at[pl.ds(base, EPT2)], sall2)
    pltpu.sync_copy(dstr.at[pl.ds(base, EPT2)], dall2)
    pltpu.sync_copy(valr.at[pl.ds(base, EPT2)], vall2)

    zi = jnp.zeros((LANES,), jnp.int32)
    zf = jnp.zeros((LANES,), jnp.float32)

    def zbody(i, carry):
        cgU[pl.ds(i * LANES, LANES)] = zi
        csU[pl.ds(i * LANES, LANES)] = zi
        cvU[pl.ds(i * LANES, LANES)] = zf
        cgI[pl.ds(i * LANES, LANES)] = zi
        csI[pl.ds(i * LANES, LANES)] = zi
        cvI[pl.ds(i * LANES, LANES)] = zf
        return carry

    lax.fori_loop(0, (EPT2 + LANES) // LANES, zbody, 0)

    def round4(ci, carry):
        kU, kI = carry
        off = ci * (4 * CHUNK)
        for kk in range((4 * CHUNK) // LANES):
            sbufc[pl.ds(kk * LANES, LANES)] = sall2[pl.ds(off + kk * LANES,
                                                          LANES)]
            dbufc[pl.ds(kk * LANES, LANES)] = dall2[pl.ds(off + kk * LANES,
                                                          LANES)]
        for q in range(4):
            pltpu.async_copy(fu.at[sbufc.at[pl.ds(q * CHUNK, CHUNK)]],
                             fbU.at[pl.ds(q * CHUNK, CHUNK)], fsem)
            pltpu.async_copy(fi.at[dbufc.at[pl.ds(q * CHUNK, CHUNK)]],
                             fbI.at[pl.ds(q * CHUNK, CHUNK)], fsem)
        for q in range(4):
            pltpu.make_async_copy(
                fu.at[sbufc.at[pl.ds(q * CHUNK, CHUNK)]],
                fbU.at[pl.ds(q * CHUNK, CHUNK)], fsem).wait()
            pltpu.make_async_copy(
                fi.at[dbufc.at[pl.ds(q * CHUNK, CHUNK)]],
                fbI.at[pl.ds(q * CHUNK, CHUNK)], fsem).wait()
        for gg in range((4 * CHUNK) // LANES):
            o16 = off + gg * LANES
            sv = sall2[pl.ds(o16, LANES)]
            dv = dall2[pl.ds(o16, LANES)]
            vv = vall2[pl.ds(o16, LANES)]
            mu = fbU[pl.ds(gg * LANES, LANES)] > 0.5
            mi = fbI[pl.ds(gg * LANES, LANES)] > 0.5
            cntU = plsc.all_reduce_population_count(mu)[0]
            cntI = plsc.all_reduce_population_count(mi)[0]
            plsc.store_compressed(cgU.at[pl.ds(kU, LANES)], dv, mask=mu)
            plsc.store_compressed(csU.at[pl.ds(kU, LANES)], sv, mask=mu)
            plsc.store_compressed(cvU.at[pl.ds(kU, LANES)], vv, mask=mu)
            plsc.store_compressed(cgI.at[pl.ds(kI, LANES)], sv, mask=mi)
            plsc.store_compressed(csI.at[pl.ds(kI, LANES)], dv, mask=mi)
            plsc.store_compressed(cvI.at[pl.ds(kI, LANES)], vv, mask=mi)
            kU = kU + cntU
            kI = kI + cntI
        return (kU, kI)

    kU, kI = lax.fori_loop(0, FCH // 4, round4,
                           (jnp.int32(0), jnp.int32(0)))

    nchU = (kU + CHUNK - 1) // CHUNK
    nchI = (kI + CHUNK - 1) // CHUNK

    def wU(i, carry):
        off = i * CHUNK
        pltpu.sync_copy(cgU.at[pl.ds(off, CHUNK)],
                        ug.at[w, pl.ds(off, CHUNK)])
        pltpu.sync_copy(csU.at[pl.ds(off, CHUNK)],
                        us.at[w, pl.ds(off, CHUNK)])
        pltpu.sync_copy(cvU.at[pl.ds(off, CHUNK)],
                        uv.at[w, pl.ds(off, CHUNK)])
        return carry

    def wI(i, carry):
        off = i * CHUNK
        pltpu.sync_copy(cgI.at[pl.ds(off, CHUNK)],
                        ig.at[w, pl.ds(off, CHUNK)])
        pltpu.sync_copy(csI.at[pl.ds(off, CHUNK)],
                        isl.at[w, pl.ds(off, CHUNK)])
        pltpu.sync_copy(cvI.at[pl.ds(off, CHUNK)],
                        iv.at[w, pl.ds(off, CHUNK)])
        return carry

    lax.fori_loop(0, nchU, wU, 0)
    lax.fori_loop(0, nchI, wI, 0)
    kvec[...] = jnp.broadcast_to(kU, (LANES,))
    pltpu.sync_copy(kvec, ucnt.at[w])
    kvec[...] = jnp.broadcast_to(kI, (LANES,))
    pltpu.sync_copy(kvec, icnt.at[w])


def _sdi(*shape):
    return jax.ShapeDtypeStruct(shape, jnp.int32)


def _sds(*shape):
    return jax.ShapeDtypeStruct(shape, jnp.float32)


_filter = pl.kernel(
    _filter_body,
    out_type=[
        _sdi(NW, EPT2), _sdi(NW, EPT2), _sds(NW, EPT2), _sdi(NW, LANES),
        _sdi(NW, EPT2), _sdi(NW, EPT2), _sds(NW, EPT2), _sdi(NW, LANES),
    ],
    mesh=_mesh,
    compiler_params=pltpu.CompilerParams(
        use_tc_tiling_on_sc=False, needs_layout_passes=False),
    scratch_types=[
        pltpu.VMEM((EPT2,), jnp.int32),
        pltpu.VMEM((EPT2,), jnp.int32),
        pltpu.VMEM((EPT2,), jnp.float32),
        pltpu.VMEM((4 * CHUNK,), jnp.int32),
        pltpu.VMEM((4 * CHUNK,), jnp.int32),
        pltpu.VMEM((4 * CHUNK,), jnp.float32),
        pltpu.VMEM((4 * CHUNK,), jnp.float32),
        pltpu.VMEM((EPT2 + LANES,), jnp.int32),
        pltpu.VMEM((EPT2 + LANES,), jnp.int32),
        pltpu.VMEM((EPT2 + LANES,), jnp.float32),
        pltpu.VMEM((EPT2 + LANES,), jnp.int32),
        pltpu.VMEM((EPT2 + LANES,), jnp.int32),
        pltpu.VMEM((EPT2 + LANES,), jnp.float32),
        pltpu.VMEM((LANES,), jnp.int32),
        pltpu.SemaphoreType.DMA,
    ],
)


def _fspmm_body(tabA, gslA, sslA, vslA, cntA, tabB, gslB, sslB, vslB, cntB,
                outA, outB, gbuf, sbuf, vbuf, idxb, cntb, rows, gsem, acc):
    _fspmm_phase(tabA, gslA, sslA, vslA, cntA, outA,
                 gbuf, sbuf, vbuf, idxb, cntb, rows, gsem, acc)
    _fspmm_phase(tabB, gslB, sslB, vslB, cntB, outB,
                 gbuf, sbuf, vbuf, idxb, cntb, rows, gsem, acc)


def _fspmm_phase(tab, gsl, ssl, vsl, cnt, out,
                 gbuf, sbuf, vbuf, idxb, cntb, rows, gsem, acc):
    c = lax.axis_index("c")
    sid = lax.axis_index("s")

    zf = jnp.zeros((LANES,), jnp.float32)

    def zrow(i, carry):
        for kk in range(HALF // LANES):
            rows[i, pl.ds(kk * LANES, LANES)] = zf
        return carry

    lax.fori_loop(0, CHUNK, zrow, 0)
    r0 = sid * ROWS_PER_TILE
    NZF = ROWS_PER_TILE // CHUNK
    remf = ROWS_PER_TILE % CHUNK
    for i in range(NZF):
        pltpu.async_copy(rows, acc.at[pl.ds(r0 + i * CHUNK, CHUNK)], gsem)
    if remf:
        pltpu.async_copy(rows.at[pl.ds(0, remf)],
                         acc.at[pl.ds(r0 + NZF * CHUNK, remf)], gsem)
    for i in range(NZF):
        pltpu.make_async_copy(rows, acc.at[pl.ds(r0 + i * CHUNK, CHUNK)],
                              gsem).wait()
    if remf:
        pltpu.make_async_copy(rows.at[pl.ds(0, remf)],
                              acc.at[pl.ds(r0 + NZF * CHUNK, remf)],
                              gsem).wait()
    plsc.subcore_barrier()

    def do_slot(slot):
        pltpu.sync_copy(cnt.at[slot], cntb)
        k = cntb[...][0]
        nch = (k + CHUNK - 1) // CHUNK

        def chunk(ci, carry):
            off = ci * CHUNK
            pltpu.sync_copy(gsl.at[slot, pl.ds(off, CHUNK)], gbuf)
            pltpu.sync_copy(ssl.at[slot, pl.ds(off, CHUNK)], sbuf)
            pltpu.sync_copy(vsl.at[slot, pl.ds(off, CHUNK)], vbuf)
            for kk in range(CHUNK // LANES):
                gv = gbuf[pl.ds(kk * LANES, LANES)]
                idxb[pl.ds(kk * LANES, LANES)] = gv * 2 + c
            pltpu.async_copy(tab.at[idxb], rows, gsem).wait()

            def scale16(gg, inner):
                vv = vbuf[pl.ds(gg * LANES, LANES)]
                for j in range(LANES):
                    val = jnp.broadcast_to(vv[j], (LANES,))
                    for kk in range(HALF // LANES):
                        rows[gg * LANES + j, pl.ds(kk * LANES, LANES)] = (
                            rows[gg * LANES + j, pl.ds(kk * LANES, LANES)]
                            * val)
                return inner

            lax.fori_loop(0, CHUNK // LANES, scale16, 0)
            pltpu.sync_copy(rows, acc.at[sbuf], add=True)
            return carry

        lax.fori_loop(0, nch, chunk, 0)

    do_slot(2 * sid)
    do_slot(2 * sid + 1)
    plsc.subcore_barrier()

    pltpu.sync_copy(acc.at[pl.ds(r0, ROWS_PER_TILE)],
                    out.at[c, pl.ds(r0, ROWS_PER_TILE)])


_fspmm = pl.kernel(
    _fspmm_body,
    out_type=[jax.ShapeDtypeStruct((NC, ROWS_PAD, HALF), jnp.float32),
              jax.ShapeDtypeStruct((NC, ROWS_PAD, HALF), jnp.float32)],
    mesh=_mesh,
    compiler_params=_sc_params,
    scratch_types=[
        pltpu.VMEM((CHUNK,), jnp.int32),
        pltpu.VMEM((CHUNK,), jnp.int32),
        pltpu.VMEM((CHUNK,), jnp.float32),
        pltpu.VMEM((CHUNK,), jnp.int32),
        pltpu.VMEM((LANES,), jnp.int32),
        pltpu.VMEM((CHUNK, HALF), jnp.float32),
        pltpu.SemaphoreType.DMA,
        pltpu.VMEM_SHARED((ROWS_PAD, HALF), jnp.float32),
    ],
)


_RB = 1000
_NBLK = N_U // _RB


def _dense_body(eu0, ei0, zu0a, zu0b, zi0a, zi0b, vtp, utp,
                eu1, ei1, vt_ei0, vt_ei1, ut_eu0, ut_eu1):
    i = pl.program_id(0)
    zu = jnp.concatenate([zu0a[0], zu0b[0]], axis=1)
    zi = jnp.concatenate([zi0a[0], zi0b[0]], axis=1)
    eu0v = eu0[...]
    ei0v = ei0[...]
    eu1v = _lrelu(zu) + eu0v
    ei1v = _lrelu(zi) + ei0v
    eu1[...] = eu1v
    ei1[...] = ei1v

    @pl.when(i == 0)
    def _():
        vt_ei0[...] = jnp.zeros_like(vt_ei0)
        vt_ei1[...] = jnp.zeros_like(vt_ei1)
        ut_eu0[...] = jnp.zeros_like(ut_eu0)
        ut_eu1[...] = jnp.zeros_like(ut_eu1)

    vtv = vtp[...]
    utv = utp[...]
    f32 = jnp.float32
    dn = (((0,), (0,)), ((), ()))

    def tdot(a, b):
        return lax.dot_general(a, b, dn, preferred_element_type=f32)

    vt_ei0[...] += tdot(vtv, ei0v)
    vt_ei1[...] += tdot(vtv, ei1v)
    ut_eu0[...] += tdot(utv, eu0v)
    ut_eu1[...] += tdot(utv, eu1v)


def _dense_stage(E_u0, E_i0, zu0, zi0, vtp, utp):
    blk = lambda i: (i, 0)

    def half(h):
        return lambda i: (h, i, 0)

    return pl.pallas_call(
        _dense_body,
        grid=(_NBLK,),
        in_specs=[
            pl.BlockSpec((_RB, D), blk),
            pl.BlockSpec((_RB, D), blk),
            pl.BlockSpec((1, _RB, HALF), half(0)),
            pl.BlockSpec((1, _RB, HALF), half(1)),
            pl.BlockSpec((1, _RB, HALF), half(0)),
            pl.BlockSpec((1, _RB, HALF), half(1)),
            pl.BlockSpec((_RB, QP), lambda i: (i, 0)),
            pl.BlockSpec((_RB, QP), lambda i: (i, 0)),
        ],
        out_specs=[
            pl.BlockSpec((_RB, D), blk),
            pl.BlockSpec((_RB, D), blk),
            pl.BlockSpec((QP, D), lambda i: (0, 0)),
            pl.BlockSpec((QP, D), lambda i: (0, 0)),
            pl.BlockSpec((QP, D), lambda i: (0, 0)),
            pl.BlockSpec((QP, D), lambda i: (0, 0)),
        ],
        out_shape=[
            jax.ShapeDtypeStruct((N_U, D), jnp.float32),
            jax.ShapeDtypeStruct((N_I, D), jnp.float32),
            jax.ShapeDtypeStruct((QP, D), jnp.float32),
            jax.ShapeDtypeStruct((QP, D), jnp.float32),
            jax.ShapeDtypeStruct((QP, D), jnp.float32),
            jax.ShapeDtypeStruct((QP, D), jnp.float32),
        ],
    )(E_u0, E_i0, zu0, zu0, zi0, zi0, vtp, utp)


def _gather_body(eu0, ei0, zu0f, zi0f, zu1f, zi1f, su, nu1, nu2, sv, nv1, nv2,
                 uids, iids, pos, neg,
                 o_eu0, o_ei0p, o_ei0n,
                 o_zu0a, o_zu0b, o_zu1a, o_zu1b,
                 o_zi0pa, o_zi0pb, o_zi0na, o_zi0nb,
                 o_zi1pa, o_zi1pb, o_zi1na, o_zi1nb,
                 o_su, o_nu1, o_nu2, o_sv, o_nv1, o_nv2,
                 iu, ii, ip, ineg, ibuf, r128, r64, r16, sem):
    c = lax.axis_index("c")
    sid = lax.axis_index("s")
    wid = sid * NC + c

    @pl.when(wid < LANES)
    def _():
        b0 = wid * LANES
        pltpu.sync_copy(uids.at[pl.ds(b0, LANES)], iu)
        pltpu.sync_copy(iids.at[pl.ds(b0, LANES)], ii)
        pltpu.sync_copy(pos.at[pl.ds(b0, LANES)], ip)
        pltpu.sync_copy(neg.at[pl.ds(b0, LANES)], ineg)

        def gat(tab, idxref, dst, off):
            if off is None:
                pltpu.async_copy(tab.at[idxref], dst, sem).wait()
            else:
                ibuf[...] = idxref[...] + off
                pltpu.async_copy(tab.at[ibuf], dst, sem).wait()
            return dst

        def put(dst_hbm, src):
            pltpu.sync_copy(src, dst_hbm.at[pl.ds(b0, LANES)])

        put(o_eu0, gat(eu0, iu, r128, None))
        put(o_ei0p, gat(ei0, ip, r128, None))
        put(o_ei0n, gat(ei0, ineg, r128, None))
        put(o_zu0a, gat(zu0f, iu, r64, 0))
        put(o_zu0b, gat(zu0f, iu, r64, ROWS_PAD))
        put(o_zu1a, gat(zu1f, iu, r64, 0))
        put(o_zu1b, gat(zu1f, iu, r64, ROWS_PAD))
        put(o_zi0pa, gat(zi0f, ip, r64, 0))
        put(o_zi0pb, gat(zi0f, ip, r64, ROWS_PAD))
        put(o_zi0na, gat(zi0f, ineg, r64, 0))
        put(o_zi0nb, gat(zi0f, ineg, r64, ROWS_PAD))
        put(o_zi1pa, gat(zi1f, ip, r64, 0))
        put(o_zi1pb, gat(zi1f, ip, r64, ROWS_PAD))
        put(o_zi1na, gat(zi1f, ineg, r64, 0))
        put(o_zi1nb, gat(zi1f, ineg, r64, ROWS_PAD))
        put(o_su, gat(su, iu, r16, None))
        put(o_nu1, gat(nu1, iu, r16, None))
        put(o_nu2, gat(nu2, iu, r16, None))
        put(o_sv, gat(sv, ii, r16, None))
        put(o_nv1, gat(nv1, ii, r16, None))
        put(o_nv2, gat(nv2, ii, r16, None))


_gather = pl.kernel(
    _gather_body,
    out_type=[
        _sds(B, D), _sds(B, D), _sds(B, D),
        _sds(B, HALF), _sds(B, HALF), _sds(B, HALF), _sds(B, HALF),
        _sds(B, HALF), _sds(B, HALF), _sds(B, HALF), _sds(B, HALF),
        _sds(B, HALF), _sds(B, HALF), _sds(B, HALF), _sds(B, HALF),
        _sds(B, QP), _sds(B, QP), _sds(B, QP),
        _sds(B, QP), _sds(B, QP), _sds(B, QP),
    ],
    mesh=_mesh,
    compiler_params=_sc_params,
    scratch_types=[
        pltpu.VMEM((LANES,), jnp.int32),
        pltpu.VMEM((LANES,), jnp.int32),
        pltpu.VMEM((LANES,), jnp.int32),
        pltpu.VMEM((LANES,), jnp.int32),
        pltpu.VMEM((LANES,), jnp.int32),
        pltpu.VMEM((LANES, D), jnp.float32),
        pltpu.VMEM((LANES, HALF), jnp.float32),
        pltpu.VMEM((LANES, QP), jnp.float32),
        pltpu.SemaphoreType.DMA,
    ],
)


def _l2n(x):
    n = jnp.sqrt(jnp.sum(x * x, axis=1, keepdims=True))
    return x / jnp.maximum(n, 1e-12)


def _loss_body(eu0g, ei0p, ei0n, zu0g, zu1g, zi0p, zi0n, zi1p, zi1n,
               sug, nu1g, nu2g, svg, nv1g, nv2g,
               vt_ei0, vt_ei1, ut_eu0, ut_eu1,
               wu, wi, umask, imask, spad, out):
    f32 = jnp.float32

    u_emb = 3.0 * eu0g[...] + 2.0 * _lrelu(zu0g[...]) + _lrelu(zu1g[...])
    e_pos = 3.0 * ei0p[...] + 2.0 * _lrelu(zi0p[...]) + _lrelu(zi1p[...])
    e_neg = 3.0 * ei0n[...] + 2.0 * _lrelu(zi0n[...]) + _lrelu(zi1n[...])
    pos_scores = jnp.sum(u_emb * e_pos, axis=1)
    neg_scores = jnp.sum(u_emb * e_neg, axis=1)
    loss_r = jnp.sum(jnp.maximum(1.0 - pos_scores + neg_scores, 0.0)) / B

    sp = spad[...]
    su1 = sug[...] + jnp.sign(sug[...]) * _l2n(nu1g[...]) * EPS
    su2 = sug[...] + jnp.sign(sug[...]) * _l2n(nu2g[...]) * EPS
    sv1 = svg[...] + jnp.sign(svg[...]) * _l2n(nv1g[...]) * EPS
    sv2 = svg[...] + jnp.sign(svg[...]) * _l2n(nv2g[...]) * EPS
    su1 = su1 * sp
    su2 = su2 * sp
    sv1 = sv1 * sp
    sv2 = sv2 * sp

    loss_s = jnp.zeros((), f32)
    for l in range(L):
        vtei = vt_ei0[...] if l == 0 else vt_ei1[...]
        uteu = ut_eu0[...] if l == 0 else ut_eu1[...]
        gu1 = _lrelu(jnp.dot(su1, vtei, preferred_element_type=f32))
        gu2 = _lrelu(jnp.dot(su2, vtei, preferred_element_type=f32))
        hu1 = jnp.dot(_l2n(gu1), wu[l], preferred_element_type=f32)
        hu2 = jnp.dot(_l2n(gu2), wu[l], preferred_element_type=f32)
        pos_sc = jnp.exp(jnp.sum(hu1 * hu2, axis=1) / TEMP)
        neg_sc = jnp.sum(jnp.exp(
            jnp.dot(hu1, hu2.T, preferred_element_type=f32) / TEMP), axis=1)
        loss_s += jnp.sum(
            -jnp.log(pos_sc / (neg_sc + 1e-8) + 1e-8) * umask[l])
        gi1 = _lrelu(jnp.dot(sv1, uteu, preferred_element_type=f32))
        gi2 = _lrelu(jnp.dot(sv2, uteu, preferred_element_type=f32))
        hi1 = jnp.dot(_l2n(gi1), wi[l], preferred_element_type=f32)
        hi2 = jnp.dot(_l2n(gi2), wi[l], preferred_element_type=f32)
        pos_sc = jnp.exp(jnp.sum(hi1 * hi2, axis=1) / TEMP)
        neg_sc = jnp.sum(jnp.exp(
            jnp.dot(hi1, hi2.T, preferred_element_type=f32) / TEMP), axis=1)
        loss_s += jnp.sum(
            -jnp.log(pos_sc / (neg_sc + 1e-8) + 1e-8) * imask[l])

    loss = loss_r + LAMBDA_1 * loss_s
    lane = lax.broadcasted_iota(jnp.int32, (1, 128), 1)
    row = jnp.where(lane == 0, loss,
                    jnp.where(lane == 1, loss_r,
                              jnp.where(lane == 2, loss_s, 0.0)))
    out[...] = row


def _loss_stage(args):
    return pl.pallas_call(
        _loss_body,
        out_shape=jax.ShapeDtypeStruct((1, 128), jnp.float32),
    )(*args)


@jax.jit
def _run(E_u_0, E_i_0, svd_u, s, svd_v, ut, vt, edge_vals,
         noise_u1, noise_v1, noise_u2, noise_v2, W_u, W_i, u_mask, i_mask,
         edge_index, uids, iids, pos, neg):
    i32 = jnp.int32
    src = edge_index[0].astype(i32)
    dst = edge_index[1].astype(i32)
    padn = E_PAD - E
    zpad_i = jnp.zeros((padn,), i32)
    src_p = jnp.concatenate([src, zpad_i])
    dst_p = jnp.concatenate([dst, zpad_i])
    vals_p = jnp.concatenate([edge_vals, jnp.zeros((padn,), jnp.float32)])

    def to_b16(x):
        return x.astype(jnp.bfloat16).reshape(2 * N_U, HALF)[:, _PERM]

    zu0, zi0 = _spmm(to_b16(E_i_0), to_b16(E_u_0), dst_p, src_p, vals_p)

    vtp = jnp.zeros((N_I, QP), jnp.float32).at[:, :Q].set(vt.T)
    utp = jnp.zeros((N_U, QP), jnp.float32).at[:, :Q].set(ut.T)
    eu1, ei1, vt_ei0, vt_ei1, ut_eu0, ut_eu1 = _dense_stage(
        E_u_0, E_i_0, zu0, zi0, vtp, utp)

    uids32 = uids.astype(i32)
    iids32 = iids.astype(i32)
    pos32 = pos.astype(i32)
    neg32 = neg.astype(i32)
    flagU = jnp.zeros((ROWS_PAD,), jnp.float32).at[uids32].set(1.0)
    flagI = (jnp.zeros((ROWS_PAD,), jnp.float32)
             .at[pos32].set(1.0).at[neg32].set(1.0))
    (ug, us, uv, ucnt, ig, isl, iv, icnt) = _filter(
        flagU, flagI, src_p, dst_p, vals_p)
    zu1, zi1 = _fspmm(ei1.reshape(2 * N_I, HALF), ug, us, uv, ucnt,
                      eu1.reshape(2 * N_U, HALF), ig, isl, iv, icnt)

    def pad_q(x):
        return jnp.zeros((x.shape[0], QP), jnp.float32).at[:, :Q].set(x)

    su16 = pad_q(svd_u)
    nu1_16 = pad_q(noise_u1)
    nu2_16 = pad_q(noise_u2)
    sv16 = pad_q(svd_v)
    nv1_16 = pad_q(noise_v1)
    nv2_16 = pad_q(noise_v2)

    def flat(z):
        return z.reshape(2 * ROWS_PAD, HALF)

    (g_eu0, g_ei0p, g_ei0n,
     g_zu0a, g_zu0b, g_zu1a, g_zu1b,
     g_zi0pa, g_zi0pb, g_zi0na, g_zi0nb,
     g_zi1pa, g_zi1pb, g_zi1na, g_zi1nb,
     g_su, g_nu1, g_nu2, g_sv, g_nv1, g_nv2) = _gather(
        E_u_0, E_i_0, flat(zu0), flat(zi0), flat(zu1), flat(zi1),
        su16, nu1_16, nu2_16, sv16, nv1_16, nv2_16,
        uids32, iids32, pos32, neg32)

    def cat(a, b):
        return jnp.concatenate([a, b], axis=1)

    zu0g = cat(g_zu0a, g_zu0b)
    zu1g = cat(g_zu1a, g_zu1b)
    zi0p = cat(g_zi0pa, g_zi0pb)
    zi0n = cat(g_zi0na, g_zi0nb)
    zi1p = cat(g_zi1pa, g_zi1pb)
    zi1n = cat(g_zi1na, g_zi1nb)

    spad = jnp.zeros((1, QP), jnp.float32).at[0, :Q].set(s)
    out = _loss_stage([
        g_eu0, g_ei0p, g_ei0n, zu0g, zu1g, zi0p, zi0n, zi1p, zi1n,
        g_su, g_nu1, g_nu2, g_sv, g_nv1, g_nv2,
        vt_ei0, vt_ei1, ut_eu0, ut_eu1,
        W_u, W_i, u_mask, i_mask, spad])
    return (out[0, 0], out[0, 1], out[0, 2])


def kernel(E_u_0, E_i_0, svd_u, s, svd_v, ut, vt, edge_vals,
           noise_u1, noise_v1, noise_u2, noise_v2, W_u, W_i, u_mask, i_mask,
           edge_index, uids, iids, pos, neg):
    return _run(E_u_0, E_i_0, svd_u, s, svd_v, ut, vt, edge_vals,
                noise_u1, noise_v1, noise_u2, noise_v2, W_u, W_i,
                u_mask, i_mask, edge_index, uids, iids, pos, neg)


# speedup vs baseline: 1.0232x; 1.0232x over previous
import jax
import jax.numpy as jnp
import numpy as np
from jax import lax
from jax.experimental import pallas as pl
from jax.experimental.pallas import tpu as pltpu
from jax.experimental.pallas import tpu_sc as plsc

N_U = 25000
N_I = 25000
D = 128
Q = 5
E = 400000
L = 2
B = 256
EPS = 0.05
TEMP = 0.2
LAMBDA_1 = 0.2

NC = 2
NS = 16
LANES = 16
HALF = D // 2

CHUNK = 128
E_PAD = 409600
EDGES_PER_TILE = E_PAD // NS
CHUNKS_PER_TILE = EDGES_PER_TILE // CHUNK

ROWS_PAD = 25088
ROWS_PER_TILE = ROWS_PAD // NS
HALF_ROWS = ROWS_PER_TILE // 2

QP = 16

_PERM = np.zeros((HALF,), np.int32)
for _g in range(HALF // 32):
    for _i in range(16):
        _PERM[_g * 32 + 2 * _i] = _g * 32 + _i
        _PERM[_g * 32 + 2 * _i + 1] = _g * 32 + 16 + _i

_mesh = plsc.VectorSubcoreMesh(
    core_axis_name="c", subcore_axis_name="s", num_cores=NC, num_subcores=NS)


def _lrelu(x):
    return jnp.where(x >= 0, x, 0.5 * x)


SUP = 2560
NSUP = EDGES_PER_TILE // SUP
SCH = 80
S_CH = SUP // SCH


def _spmm_body(tab, g, s, v, out, gall, sall, vall, idxG0, idxG1, sbuf0,
               sbuf1, rG0, rG1, rS0, rS1, gsem0, gsem1, ssem0, ssem1,
               zsem, acc):
    c = lax.axis_index("c")
    sid = lax.axis_index("s")
    rG = (rG0, rG1)
    rS = (rS0, rS1)
    idxG = (idxG0, idxG1)
    sbuf = (sbuf0, sbuf1)
    gsem = (gsem0, gsem1)
    ssem = (ssem0, ssem1)

    zf = jnp.zeros((LANES,), jnp.float32)

    def zrow(i, carry):
        for kk in range(HALF // LANES):
            rS0[i, pl.ds(kk * LANES, LANES)] = zf
        return carry

    lax.fori_loop(0, SCH, zrow, 0)
    r0 = sid * ROWS_PER_TILE
    NZ = ROWS_PER_TILE // SCH
    remz = ROWS_PER_TILE % SCH
    for i in range(NZ):
        pltpu.async_copy(rS0, acc.at[pl.ds(r0 + i * SCH, SCH)], zsem)
    if remz:
        pltpu.async_copy(rS0.at[pl.ds(0, remz)],
                         acc.at[pl.ds(r0 + NZ * SCH, remz)], zsem)
    for i in range(NZ):
        pltpu.make_async_copy(rS0, acc.at[pl.ds(r0 + i * SCH, SCH)],
                              zsem).wait()
    if remz:
        pltpu.make_async_copy(rS0.at[pl.ds(0, remz)],
                              acc.at[pl.ds(r0 + NZ * SCH, remz)], zsem).wait()
    plsc.subcore_barrier()

    ebase = sid * EDGES_PER_TILE

    def mkidx(bi, cj):
        for kk in range(SCH // LANES):
            gv = gall[pl.ds(cj * SCH + kk * LANES, LANES)]
            idxG[bi][pl.ds(kk * LANES, LANES)] = gv * 2 + c

    def mksidx(bi, cj):
        for kk in range(SCH // LANES):
            sbuf[bi][pl.ds(kk * LANES, LANES)] = (
                sall[pl.ds(cj * SCH + kk * LANES, LANES)])

    def scale(bi, cj):
        def scale16(gg, inner):
            vv = vall[pl.ds(cj * SCH + gg * LANES, LANES)]
            for j in range(LANES):
                val = jnp.broadcast_to(vv[j], (LANES,))
                r = gg * LANES + j
                for kk in range(HALF // 32):
                    grp = rG[bi][r, pl.ds(kk * 32, 32)]
                    lo, hi = plsc.unpack(
                        grp, format=plsc.PackFormat.INTERLEAVED)
                    rS[bi][r, pl.ds(kk * 32, LANES)] = lo * val
                    rS[bi][r, pl.ds(kk * 32 + LANES, LANES)] = hi * val
            return inner

        lax.fori_loop(0, SCH // LANES, scale16, 0)

    def gdrain(bi):
        pltpu.make_async_copy(tab.at[pl.ds(0, SCH)], rG[bi], gsem[bi]).wait()

    def sdrain(bi):
        pltpu.make_async_copy(out.at[0, pl.ds(0, SCH)], rS[bi],
                              ssem[bi]).wait()

    def step(bi, ch, si, lookahead):
        gdrain(bi)

        @pl.when(si * S_CH + ch >= 2)
        def _():
            sdrain(bi)

        scale(bi, ch)
        mksidx(bi, ch)
        pltpu.async_copy(rS[bi], acc.at[sbuf[bi]], ssem[bi], add=True)
        if lookahead:
            mkidx(bi, ch + 2)
            pltpu.async_copy(tab.at[idxG[bi]], rG[bi], gsem[bi])

    def super_body(si, carry):
        base = ebase + si * SUP
        pltpu.sync_copy(g.at[pl.ds(base, SUP)], gall)
        pltpu.sync_copy(s.at[pl.ds(base, SUP)], sall)
        pltpu.sync_copy(v.at[pl.ds(base, SUP)], vall)

        for bi in range(2):
            mkidx(bi, bi)
            pltpu.async_copy(tab.at[idxG[bi]], rG[bi], gsem[bi])

        def pair(j2, inner):
            a = 2 * j2
            step(0, a, si, True)
            step(1, a + 1, si, True)
            return inner

        lax.fori_loop(0, S_CH // 2 - 1, pair, 0)
        step(0, S_CH - 2, si, False)
        step(1, S_CH - 1, si, False)
        return carry

    lax.fori_loop(0, NSUP, super_body, 0)
    sdrain(0)
    sdrain(1)
    plsc.subcore_barrier()

    pltpu.sync_copy(acc.at[pl.ds(r0, ROWS_PER_TILE)],
                    out.at[c, pl.ds(r0, ROWS_PER_TILE)])


_sc_params = pltpu.CompilerParams(
    use_tc_tiling_on_sc=False, needs_layout_passes=False)

_spmm = pl.kernel(
    _spmm_body,
    out_type=jax.ShapeDtypeStruct((NC, ROWS_PAD, HALF), jnp.float32),
    mesh=_mesh,
    compiler_params=_sc_params,
    scratch_types=[
        pltpu.VMEM((SUP,), jnp.int32),
        pltpu.VMEM((SUP,), jnp.int32),
        pltpu.VMEM((SUP,), jnp.float32),
        pltpu.VMEM((SCH,), jnp.int32),
        pltpu.VMEM((SCH,), jnp.int32),
        pltpu.VMEM((SCH,), jnp.int32),
        pltpu.VMEM((SCH,), jnp.int32),
        pltpu.VMEM((SCH, HALF), jnp.bfloat16),
        pltpu.VMEM((SCH, HALF), jnp.bfloat16),
        pltpu.VMEM((SCH, HALF), jnp.float32),
        pltpu.VMEM((SCH, HALF), jnp.float32),
        pltpu.SemaphoreType.DMA,
        pltpu.SemaphoreType.DMA,
        pltpu.SemaphoreType.DMA,
        pltpu.SemaphoreType.DMA,
        pltpu.SemaphoreType.DMA,
        pltpu.VMEM_SHARED((ROWS_PAD, HALF), jnp.float32),
    ],
)


NW = NC * NS
EPT2 = E_PAD // NW
FCH = EPT2 // CHUNK


def _filter_body(fu, fi, srcr, dstr, valr,
                 ug, us, uv, ucnt, ig, isl, iv, icnt,
                 sall2, dall2, vall2, sbufc, dbufc, fbU, fbI,
                 cgU, csU, cvU, cgI, csI, cvI, kvec, fsem):
    c = lax.axis_index("c")
    sid = lax.axis_index("s")
    w = sid * NC + c
    base = w * EPT2
    pltpu.sync_copy(srcr.at[pl.ds(base, EPT2)], sall2)
    pltpu.sync_copy(dstr.at[pl.ds(base, EPT2)], dall2)
    pltpu.sync_copy(valr.at[pl.ds(base, EPT2)], vall2)

    zi = jnp.zeros((LANES,), jnp.int32)
    zf = jnp.zeros((LANES,), jnp.float32)

    def zbody(i, carry):
        cgU[pl.ds(i * LANES, LANES)] = zi
        csU[pl.ds(i * LANES, LANES)] = zi
        cvU[pl.ds(i * LANES, LANES)] = zf
        cgI[pl.ds(i * LANES, LANES)] = zi
        csI[pl.ds(i * LANES, LANES)] = zi
        cvI[pl.ds(i * LANES, LANES)] = zf
        return carry

    lax.fori_loop(0, (EPT2 + LANES) // LANES, zbody, 0)

    def round4(ci, carry):
        kU, kI = carry
        off = ci * (4 * CHUNK)
        for kk in range((4 * CHUNK) // LANES):
            sbufc[pl.ds(kk * LANES, LANES)] = sall2[pl.ds(off + kk * LANES,
                                                          LANES)]
            dbufc[pl.ds(kk * LANES, LANES)] = dall2[pl.ds(off + kk * LANES,
                                                          LANES)]
        for q in range(4):
            pltpu.async_copy(fu.at[sbufc.at[pl.ds(q * CHUNK, CHUNK)]],
                             fbU.at[pl.ds(q * CHUNK, CHUNK)], fsem)
            pltpu.async_copy(fi.at[dbufc.at[pl.ds(q * CHUNK, CHUNK)]],
                             fbI.at[pl.ds(q * CHUNK, CHUNK)], fsem)
        for q in range(4):
            pltpu.make_async_copy(
                fu.at[sbufc.at[pl.ds(q * CHUNK, CHUNK)]],
                fbU.at[pl.ds(q * CHUNK, CHUNK)], fsem).wait()
            pltpu.make_async_copy(
                fi.at[dbufc.at[pl.ds(q * CHUNK, CHUNK)]],
                fbI.at[pl.ds(q * CHUNK, CHUNK)], fsem).wait()
        for gg in range((4 * CHUNK) // LANES):
            o16 = off + gg * LANES
            sv = sall2[pl.ds(o16, LANES)]
            dv = dall2[pl.ds(o16, LANES)]
            vv = vall2[pl.ds(o16, LANES)]
            mu = fbU[pl.ds(gg * LANES, LANES)] > 0.5
            mi = fbI[pl.ds(gg * LANES, LANES)] > 0.5
            cntU = plsc.all_reduce_population_count(mu)[0]
            cntI = plsc.all_reduce_population_count(mi)[0]
            plsc.store_compressed(cgU.at[pl.ds(kU, LANES)], dv, mask=mu)
            plsc.store_compressed(csU.at[pl.ds(kU, LANES)], sv, mask=mu)
            plsc.store_compressed(cvU.at[pl.ds(kU, LANES)], vv, mask=mu)
            plsc.store_compressed(cgI.at[pl.ds(kI, LANES)], sv, mask=mi)
            plsc.store_compressed(csI.at[pl.ds(kI, LANES)], dv, mask=mi)
            plsc.store_compressed(cvI.at[pl.ds(kI, LANES)], vv, mask=mi)
            kU = kU + cntU
            kI = kI + cntI
        return (kU, kI)

    kU, kI = lax.fori_loop(0, FCH // 4, round4,
                           (jnp.int32(0), jnp.int32(0)))

    nchU = (kU + CHUNK - 1) // CHUNK
    nchI = (kI + CHUNK - 1) // CHUNK

    def wU(i, carry):
        off = i * CHUNK
        pltpu.sync_copy(cgU.at[pl.ds(off, CHUNK)],
                        ug.at[w, pl.ds(off, CHUNK)])
        pltpu.sync_copy(csU.at[pl.ds(off, CHUNK)],
                        us.at[w, pl.ds(off, CHUNK)])
        pltpu.sync_copy(cvU.at[pl.ds(off, CHUNK)],
                        uv.at[w, pl.ds(off, CHUNK)])
        return carry

    def wI(i, carry):
        off = i * CHUNK
        pltpu.sync_copy(cgI.at[pl.ds(off, CHUNK)],
                        ig.at[w, pl.ds(off, CHUNK)])
        pltpu.sync_copy(csI.at[pl.ds(off, CHUNK)],
                        isl.at[w, pl.ds(off, CHUNK)])
        pltpu.sync_copy(cvI.at[pl.ds(off, CHUNK)],
                        iv.at[w, pl.ds(off, CHUNK)])
        return carry

    lax.fori_loop(0, nchU, wU, 0)
    lax.fori_loop(0, nchI, wI, 0)
    kvec[...] = jnp.broadcast_to(kU, (LANES,))
    pltpu.sync_copy(kvec, ucnt.at[w])
    kvec[...] = jnp.broadcast_to(kI, (LANES,))
    pltpu.sync_copy(kvec, icnt.at[w])


def _sdi(*shape):
    return jax.ShapeDtypeStruct(shape, jnp.int32)


def _sds(*shape):
    return jax.ShapeDtypeStruct(shape, jnp.float32)


_filter = pl.kernel(
    _filter_body,
    out_type=[
        _sdi(NW, EPT2), _sdi(NW, EPT2), _sds(NW, EPT2), _sdi(NW, LANES),
        _sdi(NW, EPT2), _sdi(NW, EPT2), _sds(NW, EPT2), _sdi(NW, LANES),
    ],
    mesh=_mesh,
    compiler_params=pltpu.CompilerParams(
        use_tc_tiling_on_sc=False, needs_layout_passes=False),
    scratch_types=[
        pltpu.VMEM((EPT2,), jnp.int32),
        pltpu.VMEM((EPT2,), jnp.int32),
        pltpu.VMEM((EPT2,), jnp.float32),
        pltpu.VMEM((4 * CHUNK,), jnp.int32),
        pltpu.VMEM((4 * CHUNK,), jnp.int32),
        pltpu.VMEM((4 * CHUNK,), jnp.float32),
        pltpu.VMEM((4 * CHUNK,), jnp.float32),
        pltpu.VMEM((EPT2 + LANES,), jnp.int32),
        pltpu.VMEM((EPT2 + LANES,), jnp.int32),
        pltpu.VMEM((EPT2 + LANES,), jnp.float32),
        pltpu.VMEM((EPT2 + LANES,), jnp.int32),
        pltpu.VMEM((EPT2 + LANES,), jnp.int32),
        pltpu.VMEM((EPT2 + LANES,), jnp.float32),
        pltpu.VMEM((LANES,), jnp.int32),
        pltpu.SemaphoreType.DMA,
    ],
)


def _fspmm_body(tab, gsl, ssl, vsl, cnt, out,
                gbuf, sbuf, vbuf, idxb, cntb, rows, gsem, acc):
    c = lax.axis_index("c")
    sid = lax.axis_index("s")

    zf = jnp.zeros((LANES,), jnp.float32)

    def zrow(i, carry):
        for kk in range(HALF // LANES):
            rows[i, pl.ds(kk * LANES, LANES)] = zf
        return carry

    lax.fori_loop(0, CHUNK, zrow, 0)
    r0 = sid * ROWS_PER_TILE
    NZF = ROWS_PER_TILE // CHUNK
    remf = ROWS_PER_TILE % CHUNK
    for i in range(NZF):
        pltpu.async_copy(rows, acc.at[pl.ds(r0 + i * CHUNK, CHUNK)], gsem)
    if remf:
        pltpu.async_copy(rows.at[pl.ds(0, remf)],
                         acc.at[pl.ds(r0 + NZF * CHUNK, remf)], gsem)
    for i in range(NZF):
        pltpu.make_async_copy(rows, acc.at[pl.ds(r0 + i * CHUNK, CHUNK)],
                              gsem).wait()
    if remf:
        pltpu.make_async_copy(rows.at[pl.ds(0, remf)],
                              acc.at[pl.ds(r0 + NZF * CHUNK, remf)],
                              gsem).wait()
    plsc.subcore_barrier()

    def do_slot(slot):
        pltpu.sync_copy(cnt.at[slot], cntb)
        k = cntb[...][0]
        nch = (k + CHUNK - 1) // CHUNK

        def chunk(ci, carry):
            off = ci * CHUNK
            pltpu.sync_copy(gsl.at[slot, pl.ds(off, CHUNK)], gbuf)
            pltpu.sync_copy(ssl.at[slot, pl.ds(off, CHUNK)], sbuf)
            pltpu.sync_copy(vsl.at[slot, pl.ds(off, CHUNK)], vbuf)
            for kk in range(CHUNK // LANES):
                gv = gbuf[pl.ds(kk * LANES, LANES)]
                idxb[pl.ds(kk * LANES, LANES)] = gv * 2 + c
            pltpu.async_copy(tab.at[idxb], rows, gsem).wait()

            def scale16(gg, inner):
                vv = vbuf[pl.ds(gg * LANES, LANES)]
                for j in range(LANES):
                    val = jnp.broadcast_to(vv[j], (LANES,))
                    for kk in range(HALF // LANES):
                        rows[gg * LANES + j, pl.ds(kk * LANES, LANES)] = (
                            rows[gg * LANES + j, pl.ds(kk * LANES, LANES)]
                            * val)
                return inner

            lax.fori_loop(0, CHUNK // LANES, scale16, 0)
            pltpu.sync_copy(rows, acc.at[sbuf], add=True)
            return carry

        lax.fori_loop(0, nch, chunk, 0)

    do_slot(2 * sid)
    do_slot(2 * sid + 1)
    plsc.subcore_barrier()

    pltpu.sync_copy(acc.at[pl.ds(r0, ROWS_PER_TILE)],
                    out.at[c, pl.ds(r0, ROWS_PER_TILE)])


_fspmm = pl.kernel(
    _fspmm_body,
    out_type=jax.ShapeDtypeStruct((NC, ROWS_PAD, HALF), jnp.float32),
    mesh=_mesh,
    compiler_params=_sc_params,
    scratch_types=[
        pltpu.VMEM((CHUNK,), jnp.int32),
        pltpu.VMEM((CHUNK,), jnp.int32),
        pltpu.VMEM((CHUNK,), jnp.float32),
        pltpu.VMEM((CHUNK,), jnp.int32),
        pltpu.VMEM((LANES,), jnp.int32),
        pltpu.VMEM((CHUNK, HALF), jnp.float32),
        pltpu.SemaphoreType.DMA,
        pltpu.VMEM_SHARED((ROWS_PAD, HALF), jnp.float32),
    ],
)


_RB = 1000
_NBLK = N_U // _RB


def _dense_body(eu0, ei0, zu0a, zu0b, zi0a, zi0b, vtp, utp,
                eu1, ei1, vt_ei0, vt_ei1, ut_eu0, ut_eu1):
    i = pl.program_id(0)
    zu = jnp.concatenate([zu0a[0], zu0b[0]], axis=1)
    zi = jnp.concatenate([zi0a[0], zi0b[0]], axis=1)
    eu0v = eu0[...]
    ei0v = ei0[...]
    eu1v = _lrelu(zu) + eu0v
    ei1v = _lrelu(zi) + ei0v
    eu1[...] = eu1v
    ei1[...] = ei1v

    @pl.when(i == 0)
    def _():
        vt_ei0[...] = jnp.zeros_like(vt_ei0)
        vt_ei1[...] = jnp.zeros_like(vt_ei1)
        ut_eu0[...] = jnp.zeros_like(ut_eu0)
        ut_eu1[...] = jnp.zeros_like(ut_eu1)

    vtv = vtp[...]
    utv = utp[...]
    f32 = jnp.float32
    dn = (((0,), (0,)), ((), ()))

    def tdot(a, b):
        return lax.dot_general(a, b, dn, preferred_element_type=f32)

    vt_ei0[...] += tdot(vtv, ei0v)
    vt_ei1[...] += tdot(vtv, ei1v)
    ut_eu0[...] += tdot(utv, eu0v)
    ut_eu1[...] += tdot(utv, eu1v)


def _dense_stage(E_u0, E_i0, zu0, zi0, vtp, utp):
    blk = lambda i: (i, 0)

    def half(h):
        return lambda i: (h, i, 0)

    return pl.pallas_call(
        _dense_body,
        grid=(_NBLK,),
        in_specs=[
            pl.BlockSpec((_RB, D), blk),
            pl.BlockSpec((_RB, D), blk),
            pl.BlockSpec((1, _RB, HALF), half(0)),
            pl.BlockSpec((1, _RB, HALF), half(1)),
            pl.BlockSpec((1, _RB, HALF), half(0)),
            pl.BlockSpec((1, _RB, HALF), half(1)),
            pl.BlockSpec((_RB, QP), lambda i: (i, 0)),
            pl.BlockSpec((_RB, QP), lambda i: (i, 0)),
        ],
        out_specs=[
            pl.BlockSpec((_RB, D), blk),
            pl.BlockSpec((_RB, D), blk),
            pl.BlockSpec((QP, D), lambda i: (0, 0)),
            pl.BlockSpec((QP, D), lambda i: (0, 0)),
            pl.BlockSpec((QP, D), lambda i: (0, 0)),
            pl.BlockSpec((QP, D), lambda i: (0, 0)),
        ],
        out_shape=[
            jax.ShapeDtypeStruct((N_U, D), jnp.float32),
            jax.ShapeDtypeStruct((N_I, D), jnp.float32),
            jax.ShapeDtypeStruct((QP, D), jnp.float32),
            jax.ShapeDtypeStruct((QP, D), jnp.float32),
            jax.ShapeDtypeStruct((QP, D), jnp.float32),
            jax.ShapeDtypeStruct((QP, D), jnp.float32),
        ],
    )(E_u0, E_i0, zu0, zu0, zi0, zi0, vtp, utp)


def _gather_body(eu0, ei0, zu0f, zi0f, zu1f, zi1f, su, nu1, nu2, sv, nv1, nv2,
                 uids, iids, pos, neg,
                 o_eu0, o_ei0p, o_ei0n,
                 o_zu0a, o_zu0b, o_zu1a, o_zu1b,
                 o_zi0pa, o_zi0pb, o_zi0na, o_zi0nb,
                 o_zi1pa, o_zi1pb, o_zi1na, o_zi1nb,
                 o_su, o_nu1, o_nu2, o_sv, o_nv1, o_nv2,
                 iu, ii, ip, ineg, ibuf, r128, r64, r16, sem):
    c = lax.axis_index("c")
    sid = lax.axis_index("s")
    wid = sid * NC + c

    @pl.when(wid < LANES)
    def _():
        b0 = wid * LANES
        pltpu.sync_copy(uids.at[pl.ds(b0, LANES)], iu)
        pltpu.sync_copy(iids.at[pl.ds(b0, LANES)], ii)
        pltpu.sync_copy(pos.at[pl.ds(b0, LANES)], ip)
        pltpu.sync_copy(neg.at[pl.ds(b0, LANES)], ineg)

        def gat(tab, idxref, dst, off):
            if off is None:
                pltpu.async_copy(tab.at[idxref], dst, sem).wait()
            else:
                ibuf[...] = idxref[...] + off
                pltpu.async_copy(tab.at[ibuf], dst, sem).wait()
            return dst

        def put(dst_hbm, src):
            pltpu.sync_copy(src, dst_hbm.at[pl.ds(b0, LANES)])

        put(o_eu0, gat(eu0, iu, r128, None))
        put(o_ei0p, gat(ei0, ip, r128, None))
        put(o_ei0n, gat(ei0, ineg, r128, None))
        put(o_zu0a, gat(zu0f, iu, r64, 0))
        put(o_zu0b, gat(zu0f, iu, r64, ROWS_PAD))
        put(o_zu1a, gat(zu1f, iu, r64, 0))
        put(o_zu1b, gat(zu1f, iu, r64, ROWS_PAD))
        put(o_zi0pa, gat(zi0f, ip, r64, 0))
        put(o_zi0pb, gat(zi0f, ip, r64, ROWS_PAD))
        put(o_zi0na, gat(zi0f, ineg, r64, 0))
        put(o_zi0nb, gat(zi0f, ineg, r64, ROWS_PAD))
        put(o_zi1pa, gat(zi1f, ip, r64, 0))
        put(o_zi1pb, gat(zi1f, ip, r64, ROWS_PAD))
        put(o_zi1na, gat(zi1f, ineg, r64, 0))
        put(o_zi1nb, gat(zi1f, ineg, r64, ROWS_PAD))
        put(o_su, gat(su, iu, r16, None))
        put(o_nu1, gat(nu1, iu, r16, None))
        put(o_nu2, gat(nu2, iu, r16, None))
        put(o_sv, gat(sv, ii, r16, None))
        put(o_nv1, gat(nv1, ii, r16, None))
        put(o_nv2, gat(nv2, ii, r16, None))


_gather = pl.kernel(
    _gather_body,
    out_type=[
        _sds(B, D), _sds(B, D), _sds(B, D),
        _sds(B, HALF), _sds(B, HALF), _sds(B, HALF), _sds(B, HALF),
        _sds(B, HALF), _sds(B, HALF), _sds(B, HALF), _sds(B, HALF),
        _sds(B, HALF), _sds(B, HALF), _sds(B, HALF), _sds(B, HALF),
        _sds(B, QP), _sds(B, QP), _sds(B, QP),
        _sds(B, QP), _sds(B, QP), _sds(B, QP),
    ],
    mesh=_mesh,
    compiler_params=_sc_params,
    scratch_types=[
        pltpu.VMEM((LANES,), jnp.int32),
        pltpu.VMEM((LANES,), jnp.int32),
        pltpu.VMEM((LANES,), jnp.int32),
        pltpu.VMEM((LANES,), jnp.int32),
        pltpu.VMEM((LANES,), jnp.int32),
        pltpu.VMEM((LANES, D), jnp.float32),
        pltpu.VMEM((LANES, HALF), jnp.float32),
        pltpu.VMEM((LANES, QP), jnp.float32),
        pltpu.SemaphoreType.DMA,
    ],
)


def _l2n(x):
    n = jnp.sqrt(jnp.sum(x * x, axis=1, keepdims=True))
    return x / jnp.maximum(n, 1e-12)


def _loss_body(eu0g, ei0p, ei0n, zu0g, zu1g, zi0p, zi0n, zi1p, zi1n,
               sug, nu1g, nu2g, svg, nv1g, nv2g,
               vt_ei0, vt_ei1, ut_eu0, ut_eu1,
               wu, wi, umask, imask, spad, out):
    f32 = jnp.float32

    u_emb = 3.0 * eu0g[...] + 2.0 * _lrelu(zu0g[...]) + _lrelu(zu1g[...])
    e_pos = 3.0 * ei0p[...] + 2.0 * _lrelu(zi0p[...]) + _lrelu(zi1p[...])
    e_neg = 3.0 * ei0n[...] + 2.0 * _lrelu(zi0n[...]) + _lrelu(zi1n[...])
    pos_scores = jnp.sum(u_emb * e_pos, axis=1)
    neg_scores = jnp.sum(u_emb * e_neg, axis=1)
    loss_r = jnp.sum(jnp.maximum(1.0 - pos_scores + neg_scores, 0.0)) / B

    sp = spad[...]
    su1 = sug[...] + jnp.sign(sug[...]) * _l2n(nu1g[...]) * EPS
    su2 = sug[...] + jnp.sign(sug[...]) * _l2n(nu2g[...]) * EPS
    sv1 = svg[...] + jnp.sign(svg[...]) * _l2n(nv1g[...]) * EPS
    sv2 = svg[...] + jnp.sign(svg[...]) * _l2n(nv2g[...]) * EPS
    su1 = su1 * sp
    su2 = su2 * sp
    sv1 = sv1 * sp
    sv2 = sv2 * sp

    loss_s = jnp.zeros((), f32)
    for l in range(L):
        vtei = vt_ei0[...] if l == 0 else vt_ei1[...]
        uteu = ut_eu0[...] if l == 0 else ut_eu1[...]
        gu1 = _lrelu(jnp.dot(su1, vtei, preferred_element_type=f32))
        gu2 = _lrelu(jnp.dot(su2, vtei, preferred_element_type=f32))
        hu1 = jnp.dot(_l2n(gu1), wu[l], preferred_element_type=f32)
        hu2 = jnp.dot(_l2n(gu2), wu[l], preferred_element_type=f32)
        pos_sc = jnp.exp(jnp.sum(hu1 * hu2, axis=1) / TEMP)
        neg_sc = jnp.sum(jnp.exp(
            jnp.dot(hu1, hu2.T, preferred_element_type=f32) / TEMP), axis=1)
        loss_s += jnp.sum(
            -jnp.log(pos_sc / (neg_sc + 1e-8) + 1e-8) * umask[l])
        gi1 = _lrelu(jnp.dot(sv1, uteu, preferred_element_type=f32))
        gi2 = _lrelu(jnp.dot(sv2, uteu, preferred_element_type=f32))
        hi1 = jnp.dot(_l2n(gi1), wi[l], preferred_element_type=f32)
        hi2 = jnp.dot(_l2n(gi2), wi[l], preferred_element_type=f32)
        pos_sc = jnp.exp(jnp.sum(hi1 * hi2, axis=1) / TEMP)
        neg_sc = jnp.sum(jnp.exp(
            jnp.dot(hi1, hi2.T, preferred_element_type=f32) / TEMP), axis=1)
        loss_s += jnp.sum(
            -jnp.log(pos_sc / (neg_sc + 1e-8) + 1e-8) * imask[l])

    loss = loss_r + LAMBDA_1 * loss_s
    lane = lax.broadcasted_iota(jnp.int32, (1, 128), 1)
    row = jnp.where(lane == 0, loss,
                    jnp.where(lane == 1, loss_r,
                              jnp.where(lane == 2, loss_s, 0.0)))
    out[...] = row


def _loss_stage(args):
    return pl.pallas_call(
        _loss_body,
        out_shape=jax.ShapeDtypeStruct((1, 128), jnp.float32),
    )(*args)


@jax.jit
def _run(E_u_0, E_i_0, svd_u, s, svd_v, ut, vt, edge_vals,
         noise_u1, noise_v1, noise_u2, noise_v2, W_u, W_i, u_mask, i_mask,
         edge_index, uids, iids, pos, neg):
    i32 = jnp.int32
    src = edge_index[0].astype(i32)
    dst = edge_index[1].astype(i32)
    padn = E_PAD - E
    zpad_i = jnp.zeros((padn,), i32)
    src_p = jnp.concatenate([src, zpad_i])
    dst_p = jnp.concatenate([dst, zpad_i])
    vals_p = jnp.concatenate([edge_vals, jnp.zeros((padn,), jnp.float32)])

    def to_b16(x):
        return x.astype(jnp.bfloat16).reshape(2 * N_U, HALF)[:, _PERM]

    zu0 = _spmm(to_b16(E_i_0), dst_p, src_p, vals_p)
    zi0 = _spmm(to_b16(E_u_0), src_p, dst_p, vals_p)

    vtp = jnp.zeros((N_I, QP), jnp.float32).at[:, :Q].set(vt.T)
    utp = jnp.zeros((N_U, QP), jnp.float32).at[:, :Q].set(ut.T)
    eu1, ei1, vt_ei0, vt_ei1, ut_eu0, ut_eu1 = _dense_stage(
        E_u_0, E_i_0, zu0, zi0, vtp, utp)

    uids32 = uids.astype(i32)
    iids32 = iids.astype(i32)
    pos32 = pos.astype(i32)
    neg32 = neg.astype(i32)
    flagU = jnp.zeros((ROWS_PAD,), jnp.float32).at[uids32].set(1.0)
    flagI = (jnp.zeros((ROWS_PAD,), jnp.float32)
             .at[pos32].set(1.0).at[neg32].set(1.0))
    (ug, us, uv, ucnt, ig, isl, iv, icnt) = _filter(
        flagU, flagI, src_p, dst_p, vals_p)
    zu1 = _fspmm(ei1.reshape(2 * N_I, HALF), ug, us, uv, ucnt)
    zi1 = _fspmm(eu1.reshape(2 * N_U, HALF), ig, isl, iv, icnt)

    def pad_q(x):
        return jnp.zeros((x.shape[0], QP), jnp.float32).at[:, :Q].set(x)

    su16 = pad_q(svd_u)
    nu1_16 = pad_q(noise_u1)
    nu2_16 = pad_q(noise_u2)
    sv16 = pad_q(svd_v)
    nv1_16 = pad_q(noise_v1)
    nv2_16 = pad_q(noise_v2)

    def flat(z):
        return z.reshape(2 * ROWS_PAD, HALF)

    (g_eu0, g_ei0p, g_ei0n,
     g_zu0a, g_zu0b, g_zu1a, g_zu1b,
     g_zi0pa, g_zi0pb, g_zi0na, g_zi0nb,
     g_zi1pa, g_zi1pb, g_zi1na, g_zi1nb,
     g_su, g_nu1, g_nu2, g_sv, g_nv1, g_nv2) = _gather(
        E_u_0, E_i_0, flat(zu0), flat(zi0), flat(zu1), flat(zi1),
        su16, nu1_16, nu2_16, sv16, nv1_16, nv2_16,
        uids32, iids32, pos32, neg32)

    def cat(a, b):
        return jnp.concatenate([a, b], axis=1)

    zu0g = cat(g_zu0a, g_zu0b)
    zu1g = cat(g_zu1a, g_zu1b)
    zi0p = cat(g_zi0pa, g_zi0pb)
    zi0n = cat(g_zi0na, g_zi0nb)
    zi1p = cat(g_zi1pa, g_zi1pb)
    zi1n = cat(g_zi1na, g_zi1nb)

    spad = jnp.zeros((1, QP), jnp.float32).at[0, :Q].set(s)
    out = _loss_stage([
        g_eu0, g_ei0p, g_ei0n, zu0g, zu1g, zi0p, zi0n, zi1p, zi1n,
        g_su, g_nu1, g_nu2, g_sv, g_nv1, g_nv2,
        vt_ei0, vt_ei1, ut_eu0, ut_eu1,
        W_u, W_i, u_mask, i_mask, spad])
    return (out[0, 0], out[0, 1], out[0, 2])


def kernel(E_u_0, E_i_0, svd_u, s, svd_v, ut, vt, edge_vals,
           noise_u1, noise_v1, noise_u2, noise_v2, W_u, W_i, u_mask, i_mask,
           edge_index, uids, iids, pos, neg):
    return _run(E_u_0, E_i_0, svd_u, s, svd_v, ut, vt, edge_vals,
                noise_u1, noise_v1, noise_u2, noise_v2, W_u, W_i,
                u_mask, i_mask, edge_index, uids, iids, pos, neg)
